# Initial kernel scaffold; baseline (speedup 1.0000x reference)
import functools

import jax
import jax.numpy as jnp
from jax import lax
from jax.experimental import pallas as pl
from jax.experimental.pallas import tpu as pltpu
from jax.experimental.pallas import tpu_sc as plsc

ENT_VOCAB = 1000000
REL_VOCAB = 1000
DIM = 64
BATCH = 16384
HIST = 50
NROWS = BATCH * HIST

NC = 2
NS = 16
NW = NC * NS
ROWS_PER_W = NROWS // NW
CHUNK = 128
NCHUNK = ROWS_PER_W // CHUNK
NBUF = 4


def _run_table(tab_hbm, idx_v, out_hbm, base, buf, gsem):
    def start_gather(j, b):
        pltpu.async_copy(tab_hbm.at[idx_v.at[j]], buf.at[b], gsem)

    def wait_gather(b):
        pltpu.make_async_copy(tab_hbm.at[idx_v.at[0]], buf.at[b], gsem).wait()

    def store(j, b):
        pltpu.sync_copy(buf.at[b], out_hbm.at[pl.ds(base + j * CHUNK, CHUNK)])

    for b in range(NBUF):
        start_gather(b, b)

    def body(g, carry):
        for b in range(NBUF):
            j = g * NBUF + b
            wait_gather(b)
            store(j, b)
            start_gather(j + NBUF, b)
        return carry

    lax.fori_loop(0, NCHUNK // NBUF - 1, body, 0)

    for b in range(NBUF):
        j = NCHUNK - NBUF + b
        wait_gather(b)
        store(j, b)


def _emb_kernel(eidx_hbm, ridx_hbm, ent_hbm, rel_hbm, eout, rout,
                eidx_v, ridx_v, buf, gsem):
    wid = lax.axis_index("s") * NC + lax.axis_index("c")
    base = wid * ROWS_PER_W
    pltpu.sync_copy(eidx_hbm.at[wid], eidx_v)
    pltpu.sync_copy(ridx_hbm.at[wid], ridx_v)
    _run_table(ent_hbm, eidx_v, eout, base, buf, gsem)
    _run_table(rel_hbm, ridx_v, rout, base, buf, gsem)


@jax.jit
def _lookup(eidx, ridx, ent_table, rel_table):
    out_t = jax.ShapeDtypeStruct((NROWS, DIM), jnp.float32)
    k = functools.partial(
        pl.kernel,
        out_type=[out_t, out_t],
        mesh=plsc.VectorSubcoreMesh(core_axis_name="c", subcore_axis_name="s"),
        scratch_types=[
            pltpu.VMEM((NCHUNK, CHUNK), jnp.int32),
            pltpu.VMEM((NCHUNK, CHUNK), jnp.int32),
            pltpu.VMEM((NBUF, CHUNK, DIM), jnp.float32),
            pltpu.SemaphoreType.DMA,
        ],
    )(_emb_kernel)
    return k(eidx, ridx, ent_table, rel_table)


def kernel(entity_indices, relation_indices, entity_table, relation_table):
    b, h = entity_indices.shape
    eidx = entity_indices.reshape(NW, NCHUNK, CHUNK)
    ridx = relation_indices.reshape(NW, NCHUNK, CHUNK)
    eout, rout = _lookup(eidx, ridx, entity_table, relation_table)
    return (eout.reshape(b, h, DIM), rout.reshape(b, h, DIM))


# speedup vs baseline: 3.1647x; 3.1647x over previous
import functools

import jax
import jax.numpy as jnp
from jax import lax
from jax.experimental import pallas as pl
from jax.experimental.pallas import tpu as pltpu
from jax.experimental.pallas import tpu_sc as plsc

ENT_VOCAB = 1000000
REL_VOCAB = 1000
DIM = 64
BATCH = 16384
HIST = 50
NROWS = BATCH * HIST

NC = 2
NS = 16
NW = NC * NS
ROWS_PER_W = NROWS // NW
CHUNK = 128
NCHUNK = ROWS_PER_W // CHUNK
NBUF = 4


def _run_table(tab_hbm, idx_v, out_hbm, base, buf, gsem):
    def start_gather(j, b):
        pltpu.async_copy(tab_hbm.at[idx_v.at[j]], buf.at[b], gsem)

    def wait_gather(b):
        pltpu.make_async_copy(tab_hbm.at[idx_v.at[0]], buf.at[b], gsem).wait()

    def store(j, b):
        pltpu.sync_copy(buf.at[b], out_hbm.at[pl.ds(base + j * CHUNK, CHUNK)])

    for b in range(NBUF):
        start_gather(b, b)

    def body(g, carry):
        for b in range(NBUF):
            j = g * NBUF + b
            wait_gather(b)
            store(j, b)
            start_gather(j + NBUF, b)
        return carry

    lax.fori_loop(0, NCHUNK // NBUF - 1, body, 0)

    for b in range(NBUF):
        j = NCHUNK - NBUF + b
        wait_gather(b)
        store(j, b)


def _emb_kernel(eidx_hbm, ridx_hbm, ent_hbm, rel_hbm, eout, rout,
                eidx_v, ridx_v, buf, gsem):
    wid = lax.axis_index("s") * NC + lax.axis_index("c")
    base = wid * ROWS_PER_W
    pltpu.sync_copy(eidx_hbm.at[wid], eidx_v)
    pltpu.sync_copy(ridx_hbm.at[wid], ridx_v)
    _run_table(ent_hbm, eidx_v, eout, base, buf, gsem)
    _run_table(rel_hbm, ridx_v, rout, base, buf, gsem)


@jax.jit
def _lookup(eidx, ridx, ent_table, rel_table):
    out_t = jax.ShapeDtypeStruct((NROWS, DIM), jnp.float32)
    k = functools.partial(
        pl.kernel,
        out_type=[out_t, out_t],
        mesh=plsc.VectorSubcoreMesh(core_axis_name="c", subcore_axis_name="s"),
        compiler_params=pltpu.CompilerParams(use_tc_tiling_on_sc=False),
        scratch_types=[
            pltpu.VMEM((NCHUNK, CHUNK), jnp.int32),
            pltpu.VMEM((NCHUNK, CHUNK), jnp.int32),
            pltpu.VMEM((NBUF, CHUNK, DIM), jnp.float32),
            pltpu.SemaphoreType.DMA,
        ],
    )(_emb_kernel)
    return k(eidx, ridx, ent_table, rel_table)


def kernel(entity_indices, relation_indices, entity_table, relation_table):
    b, h = entity_indices.shape
    eidx = entity_indices.reshape(NW, NCHUNK, CHUNK)
    ridx = relation_indices.reshape(NW, NCHUNK, CHUNK)
    eout, rout = _lookup(eidx, ridx, entity_table, relation_table)
    return (eout.reshape(b, h, DIM), rout.reshape(b, h, DIM))
